# baseline (device time: 17133 ns/iter reference)
import jax
import jax.numpy as jnp
from jax import lax
from jax.experimental import pallas as pl
from jax.experimental.pallas import tpu as pltpu

K = 8


def kernel(partial, resid, gamma):
    _, m, d = partial.shape
    half = m // 2
    rpc = half // K
    gamma2d = gamma.reshape(1, d)

    def body(partial_ref, resid_ref, gamma_ref, out_ref,
             pbuf, rbuf, xbuf, obuf,
             local_sems, xsend_sems, xrecv_sems, ysend_sems, yrecv_sems,
             osend_sems):
        my_x = lax.axis_index("x")
        my_y = lax.axis_index("y")
        my_z = lax.axis_index("z")
        xpeer = (1 - my_x, my_y, my_z)
        ypeer = (my_x, my_y ^ 1, my_z)

        h = my_y % 2
        base = h * half

        barrier_sem = pltpu.get_barrier_semaphore()
        for peer in (xpeer, ypeer):
            pl.semaphore_signal(
                barrier_sem, inc=1,
                device_id=peer, device_id_type=pl.DeviceIdType.MESH,
            )
        pl.semaphore_wait(barrier_sem, 2)

        xrdmas = []
        for k in range(K):
            rows = pl.ds(base + k * rpc, rpc)
            rdma = pltpu.make_async_remote_copy(
                src_ref=partial_ref.at[0, rows, :],
                dst_ref=xbuf.at[k],
                send_sem=xsend_sems.at[k],
                recv_sem=xrecv_sems.at[k],
                device_id=xpeer,
                device_id_type=pl.DeviceIdType.MESH,
            )
            rdma.start()
            xrdmas.append(rdma)

        myrows = pl.ds(base, half)
        pcopy = pltpu.make_async_copy(
            partial_ref.at[0, myrows, :], pbuf, local_sems.at[0])
        pcopy.start()
        rcopy = pltpu.make_async_copy(
            resid_ref.at[myrows, :], rbuf, local_sems.at[1])
        rcopy.start()
        pcopy.wait()
        rcopy.wait()

        yrdmas = []
        ocopies = []
        for k in range(K):
            xrdmas[k].wait_recv()
            lrows = pl.ds(k * rpc, rpc)
            grows = pl.ds(base + k * rpc, rpc)
            y = pbuf[lrows, :] + xbuf[k, :, :] + rbuf[lrows, :]
            ms = jnp.mean(y * y, axis=-1, keepdims=True)
            obuf[lrows, :] = y * lax.rsqrt(ms + 1e-6) * gamma_ref[0, :]
            oc = pltpu.make_async_copy(
                obuf.at[lrows, :], out_ref.at[grows, :], osend_sems.at[k])
            oc.start()
            ocopies.append(oc)
            yr = pltpu.make_async_remote_copy(
                src_ref=obuf.at[lrows, :],
                dst_ref=out_ref.at[grows, :],
                send_sem=ysend_sems.at[k],
                recv_sem=yrecv_sems.at[k],
                device_id=ypeer,
                device_id_type=pl.DeviceIdType.MESH,
            )
            yr.start()
            yrdmas.append(yr)

        for k in range(K):
            yrdmas[k].wait_recv()
        for k in range(K):
            ocopies[k].wait()
            xrdmas[k].wait_send()
            yrdmas[k].wait_send()

    return pl.pallas_call(
        body,
        out_shape=jax.ShapeDtypeStruct((m, d), jnp.float32),
        in_specs=[
            pl.BlockSpec(memory_space=pl.ANY),
            pl.BlockSpec(memory_space=pl.ANY),
            pl.BlockSpec(memory_space=pltpu.VMEM),
        ],
        out_specs=pl.BlockSpec(memory_space=pl.ANY),
        scratch_shapes=[
            pltpu.VMEM((half, d), jnp.float32),
            pltpu.VMEM((half, d), jnp.float32),
            pltpu.VMEM((K, rpc, d), jnp.float32),
            pltpu.VMEM((half, d), jnp.float32),
            pltpu.SemaphoreType.DMA((2,)),
            pltpu.SemaphoreType.DMA((K,)),
            pltpu.SemaphoreType.DMA((K,)),
            pltpu.SemaphoreType.DMA((K,)),
            pltpu.SemaphoreType.DMA((K,)),
            pltpu.SemaphoreType.DMA((K,)),
        ],
        compiler_params=pltpu.CompilerParams(collective_id=0),
    )(partial, resid, gamma2d)


# device time: 14105 ns/iter; 1.2147x vs baseline; 1.2147x over previous
import jax
import jax.numpy as jnp
from jax import lax
from jax.experimental import pallas as pl
from jax.experimental.pallas import tpu as pltpu

K = 8


def kernel(partial, resid, gamma):
    _, m, d = partial.shape
    half = m // 2
    rpc = half // K
    gamma2d = gamma.reshape(1, d)

    def body(partial_ref, resid_ref, gamma_ref, out_ref,
             pbuf, rbuf, xbuf, obuf, gbuf,
             local_sems, xsend_sems, xrecv_sems, ysend_sems, yrecv_sems,
             osend_sems):
        my_x = lax.axis_index("x")
        my_y = lax.axis_index("y")
        my_z = lax.axis_index("z")
        xpeer = (1 - my_x, my_y, my_z)
        ypeer = (my_x, my_y ^ 1, my_z)

        h = my_y % 2
        base = h * half

        barrier_sem = pltpu.get_barrier_semaphore()
        for peer in (xpeer, ypeer):
            pl.semaphore_signal(
                barrier_sem, inc=1,
                device_id=peer, device_id_type=pl.DeviceIdType.MESH,
            )
        pl.semaphore_wait(barrier_sem, 2)

        xrdmas = []
        for k in range(K):
            rows = pl.ds(base + k * rpc, rpc)
            rdma = pltpu.make_async_remote_copy(
                src_ref=partial_ref.at[0, rows, :],
                dst_ref=xbuf.at[k],
                send_sem=xsend_sems.at[k],
                recv_sem=xrecv_sems.at[k],
                device_id=xpeer,
                device_id_type=pl.DeviceIdType.MESH,
            )
            rdma.start()
            xrdmas.append(rdma)

        myrows = pl.ds(base, half)
        pcopy = pltpu.make_async_copy(
            partial_ref.at[0, myrows, :], pbuf, local_sems.at[0])
        pcopy.start()
        rcopy = pltpu.make_async_copy(
            resid_ref.at[myrows, :], rbuf, local_sems.at[1])
        rcopy.start()
        gcopy = pltpu.make_async_copy(gamma_ref, gbuf, local_sems.at[2])
        gcopy.start()
        pcopy.wait()
        rcopy.wait()
        gcopy.wait()

        yrdmas = []
        ocopies = []
        for k in range(K):
            xrdmas[k].wait_recv()
            lrows = pl.ds(k * rpc, rpc)
            grows = pl.ds(base + k * rpc, rpc)
            y = pbuf[lrows, :] + xbuf[k, :, :] + rbuf[lrows, :]
            ms = jnp.mean(y * y, axis=-1, keepdims=True)
            obuf[lrows, :] = y * lax.rsqrt(ms + 1e-6) * gbuf[0, :]
            oc = pltpu.make_async_copy(
                obuf.at[lrows, :], out_ref.at[grows, :], osend_sems.at[k])
            oc.start()
            ocopies.append(oc)
            yr = pltpu.make_async_remote_copy(
                src_ref=obuf.at[lrows, :],
                dst_ref=out_ref.at[grows, :],
                send_sem=ysend_sems.at[k],
                recv_sem=yrecv_sems.at[k],
                device_id=ypeer,
                device_id_type=pl.DeviceIdType.MESH,
            )
            yr.start()
            yrdmas.append(yr)

        for k in range(K):
            yrdmas[k].wait_recv()
        for k in range(K):
            ocopies[k].wait()
            xrdmas[k].wait_send()
            yrdmas[k].wait_send()

    hbm = pltpu.MemorySpace.HBM
    return pl.pallas_call(
        body,
        out_shape=jax.ShapeDtypeStruct((m, d), jnp.float32),
        in_specs=[
            pl.BlockSpec(memory_space=hbm),
            pl.BlockSpec(memory_space=hbm),
            pl.BlockSpec(memory_space=hbm),
        ],
        out_specs=pl.BlockSpec(memory_space=hbm),
        scratch_shapes=[
            pltpu.VMEM((half, d), jnp.float32),
            pltpu.VMEM((half, d), jnp.float32),
            pltpu.VMEM((K, rpc, d), jnp.float32),
            pltpu.VMEM((half, d), jnp.float32),
            pltpu.VMEM((1, d), jnp.float32),
            pltpu.SemaphoreType.DMA((3,)),
            pltpu.SemaphoreType.DMA((K,)),
            pltpu.SemaphoreType.DMA((K,)),
            pltpu.SemaphoreType.DMA((K,)),
            pltpu.SemaphoreType.DMA((K,)),
            pltpu.SemaphoreType.DMA((K,)),
        ],
        compiler_params=pltpu.CompilerParams(collective_id=0),
    )(
        pltpu.with_memory_space_constraint(partial, hbm),
        pltpu.with_memory_space_constraint(resid, hbm),
        pltpu.with_memory_space_constraint(gamma2d, hbm),
    )
